# baseline (speedup 1.0000x reference)
import jax
import jax.numpy as jnp
from jax import lax
from jax.experimental import pallas as pl
from jax.experimental.pallas import tpu as pltpu

_B = 1024
_M = 256
_D = 128
_NCH = 16
_BPC = _B // _NCH
_RPC = _BPC * _M


def _body(state_sref, state_ref, z_ref, mem_ref, out_ref, ctr_ref,
          sem_cp, sem_row):
    ctr_ref[...] = state_ref[...] + 1

    def cp(c):
        return pltpu.make_async_copy(
            mem_ref.at[pl.ds(c * _RPC, _RPC)],
            out_ref.at[pl.ds(c * _RPC, _RPC)], sem_cp.at[c])

    for c in range(_NCH):
        cp(c).start()

    def chunk(c, _):
        cp(c).wait()

        def row(b, _):
            gb = c * _BPC + b
            r = lax.rem(state_sref[gb], _M)
            pltpu.make_async_copy(
                z_ref.at[pl.ds(gb, 1)],
                out_ref.at[pl.ds(gb * _M + r, 1)], sem_row).start()
            return 0

        return lax.fori_loop(0, _BPC, row, 0)

    lax.fori_loop(0, _NCH, chunk, 0)
    pltpu.make_async_copy(mem_ref.at[pl.ds(0, _B)], z_ref, sem_row).wait()


def kernel(z, mem_state, state):
    b, m, d = mem_state.shape
    mem2d = mem_state.reshape(b * m, d)
    state2d = state.reshape(b, 1)
    grid_spec = pltpu.PrefetchScalarGridSpec(
        num_scalar_prefetch=1,
        grid=(1,),
        in_specs=[
            pl.BlockSpec((b, 1), lambda i, s_ref: (0, 0)),
            pl.BlockSpec((b, d), lambda i, s_ref: (0, 0)),
            pl.BlockSpec(memory_space=pltpu.MemorySpace.HBM),
        ],
        out_specs=[
            pl.BlockSpec(memory_space=pltpu.MemorySpace.HBM),
            pl.BlockSpec((b, 1), lambda i, s_ref: (0, 0)),
        ],
        scratch_shapes=[pltpu.SemaphoreType.DMA((_NCH,)),
                        pltpu.SemaphoreType.DMA],
    )
    out2d, ctr2d = pl.pallas_call(
        _body,
        grid_spec=grid_spec,
        out_shape=[
            jax.ShapeDtypeStruct((b * m, d), mem_state.dtype),
            jax.ShapeDtypeStruct((b, 1), state.dtype),
        ],
    )(state, state2d, z, mem2d)
    return out2d.reshape(b, m, d), ctr2d.reshape(b)


# speedup vs baseline: 45.3230x; 45.3230x over previous
import jax
import jax.numpy as jnp
from jax import lax
from jax.experimental import pallas as pl
from jax.experimental.pallas import tpu as pltpu

_B = 1024
_M = 256
_D = 128
_CB = 32
_CROWS = _CB * _M
_NCHUNK = _B // _CB
_NBUF = 8
_PRE = 4
_QOUT = 3


def _body(state_sref, state_ref, z_ref, mem_ref, out_ref, ctr_ref,
          bufs, sem_in, sem_out):
    ctr_ref[...] = state_ref[...] + 1

    def cp_in(j, s):
        return pltpu.make_async_copy(
            mem_ref.at[pl.ds(j * _CROWS, _CROWS)], bufs.at[s], sem_in.at[s])

    def cp_out(j, s):
        return pltpu.make_async_copy(
            bufs.at[s], out_ref.at[pl.ds(j * _CROWS, _CROWS)], sem_out.at[s])

    def patch(j, s):
        for b in range(_CB):
            gb = j * _CB + b
            r = lax.rem(state_sref[gb], _M)
            bufs[s, pl.ds(b * _M + r, 1), :] = z_ref[pl.ds(gb, 1), :]

    for c in range(_PRE):
        cp_in(c, c % _NBUF).start()

    for j in range(_NCHUNK):
        s = j % _NBUF
        cp_in(j, s).wait()
        patch(j, s)
        cp_out(j, s).start()
        if j >= _QOUT:
            jq = j - _QOUT
            cp_out(jq, jq % _NBUF).wait()
        if j + _PRE < _NCHUNK:
            jn = j + _PRE
            cp_in(jn, jn % _NBUF).start()

    for q in range(_QOUT):
        j = _NCHUNK - _QOUT + q
        cp_out(j, j % _NBUF).wait()


def kernel(z, mem_state, state):
    b, m, d = mem_state.shape
    mem2d = mem_state.reshape(b * m, d)
    state2d = state.reshape(b, 1)
    grid_spec = pltpu.PrefetchScalarGridSpec(
        num_scalar_prefetch=1,
        grid=(1,),
        in_specs=[
            pl.BlockSpec((b, 1), lambda i, s_ref: (0, 0)),
            pl.BlockSpec((b, d), lambda i, s_ref: (0, 0)),
            pl.BlockSpec(memory_space=pltpu.MemorySpace.HBM),
        ],
        out_specs=[
            pl.BlockSpec(memory_space=pltpu.MemorySpace.HBM),
            pl.BlockSpec((b, 1), lambda i, s_ref: (0, 0)),
        ],
        scratch_shapes=[
            pltpu.VMEM((_NBUF, _CROWS, _D), jnp.float32),
            pltpu.SemaphoreType.DMA((_NBUF,)),
            pltpu.SemaphoreType.DMA((_NBUF,)),
        ],
    )
    out2d, ctr2d = pl.pallas_call(
        _body,
        grid_spec=grid_spec,
        out_shape=[
            jax.ShapeDtypeStruct((b * m, d), mem_state.dtype),
            jax.ShapeDtypeStruct((b, 1), state.dtype),
        ],
    )(state, state2d, z, mem2d)
    return out2d.reshape(b, m, d), ctr2d.reshape(b)
